# baseline (speedup 1.0000x reference)
import functools

import jax
import jax.numpy as jnp
from jax import lax
from jax.experimental import pallas as pl
from jax.experimental.pallas import tpu as pltpu
from jax.experimental.pallas import tpu_sc as plsc

N = 10000
E = 320000
IN_DIM = 128
NUM_HEADS = 8
HIDDEN = 16
OUT_DIM = 2

NC = 2
NS = 16
NTILES = NC * NS
EPT = E // NTILES
T = 80
NITER = EPT // T
N_PAD = 10240
ROWS = N_PAD // NS


def _pre_body(x_ref, w_ref, bs_ref, bd_ref, h_ref, al_ref, al2_ref, m_ref):
    xv = x_ref[...]
    hv = jnp.dot(xv, w_ref[...], preferred_element_type=jnp.float32)
    h_ref[...] = hv
    als = jnp.dot(hv, bs_ref[...], preferred_element_type=jnp.float32)
    ald = jnp.dot(hv, bd_ref[...], preferred_element_type=jnp.float32)
    al = jnp.concatenate([als, ald], axis=1)
    al_ref[...] = al
    al2_ref[...] = jnp.concatenate([ald, als], axis=1)
    m_ref[...] = jnp.max(al, axis=0, keepdims=True)


def _pre(x, W, Bs, Bd):
    return pl.pallas_call(
        _pre_body,
        out_shape=[
            jax.ShapeDtypeStruct((N, IN_DIM), jnp.float32),
            jax.ShapeDtypeStruct((N, 2 * NUM_HEADS), jnp.float32),
            jax.ShapeDtypeStruct((N, 2 * NUM_HEADS), jnp.float32),
            jax.ShapeDtypeStruct((1, 2 * NUM_HEADS), jnp.float32),
        ],
    )(x, W, Bs, Bd)


_GDN = lax.GatherDimensionNumbers(
    offset_dims=(), collapsed_slice_dims=(0,), start_index_map=(0,))


def _take16(v, idx):
    return lax.gather(v, idx[:, None], _GDN, (1,),
                      mode=lax.GatherScatterMode.PROMISE_IN_BOUNDS)


def _sc_body(h_hbm, al_hbm, al2_hbm, sd_hbm, shift_hbm, z128_hbm, z8_hbm,
             u_out, d_out,
             u_sh, d_sh, idxb, sdb, asb, adb, hb, wb, shb,
             si0, si1, sg0, sg1, ss0, ss1):
    c = lax.axis_index("c")
    s = lax.axis_index("s")
    tile = c * NS + s

    r0 = s * ROWS
    pltpu.sync_copy(z128_hbm.at[pl.ds(r0, ROWS)], u_sh.at[pl.ds(r0, ROWS)])
    pltpu.sync_copy(z8_hbm.at[pl.ds(r0, ROWS)], d_sh.at[pl.ds(r0, ROWS)])
    pltpu.sync_copy(shift_hbm, shb)
    plsc.subcore_barrier()

    shift = shb[...]
    sem_i = (si0, si1)
    sem_g = (sg0, sg1)
    sem_s = (ss0, ss1)
    cid0 = tile * NITER

    def issue_idx(g, b):
        pltpu.async_copy(sd_hbm.at[cid0 + g], idxb.at[b], sem_i[b])

    def wait_idx(b):
        pltpu.make_async_copy(sd_hbm.at[0], idxb.at[b], sem_i[b]).wait()

    def build_sdb(b):
        for k in range(T // 16):
            sdb[b, pl.ds(16 * k, 16)] = idxb[b, pl.ds(T + 16 * k, 16)]

    def gather_descs(b):
        srcv = idxb.at[b, pl.ds(0, T)]
        dstv = idxb.at[b, pl.ds(T, T)]
        return (
            pltpu.make_async_copy(al_hbm.at[srcv], asb.at[b], sem_g[b]),
            pltpu.make_async_copy(al2_hbm.at[dstv], adb.at[b], sem_g[b]),
            pltpu.make_async_copy(h_hbm.at[srcv], hb.at[b], sem_g[b]),
        )

    def issue_gathers(b):
        srcv = idxb.at[b, pl.ds(0, T)]
        dstv = idxb.at[b, pl.ds(T, T)]
        pltpu.async_copy(al_hbm.at[srcv], asb.at[b], sem_g[b])
        pltpu.async_copy(al2_hbm.at[dstv], adb.at[b], sem_g[b])
        pltpu.async_copy(h_hbm.at[srcv], hb.at[b], sem_g[b])

    def wait_gathers(b):
        for d in gather_descs(b):
            d.wait()

    def issue_scatters(b):
        pltpu.async_copy(hb.at[b], u_sh.at[sdb.at[b]], sem_s[b], add=True)
        pltpu.async_copy(wb.at[b], d_sh.at[sdb.at[b]], sem_s[b], add=True)

    def wait_scatters(b):
        pltpu.make_async_copy(hb.at[b], u_sh.at[sdb.at[b]], sem_s[b]).wait()
        pltpu.make_async_copy(wb.at[b], d_sh.at[sdb.at[b]], sem_s[b]).wait()

    def compute(b):
        def edge_pair(j, carry2):
            i0 = 16 * j
            ws = []
            for e in range(16):
                ra = asb[b, i0 + e, :]
                rb = adb[b, i0 + e, :]
                t = ra + rb
                t = jnp.where(t >= 0.0, t, 0.2 * t)
                w = jnp.exp(t - shift)
                wb[b, i0 + e, :] = w
                ws.append(w)
            for k in range(NUM_HEADS):
                kidx = jnp.full((16,), k, jnp.int32)
                h_i = k * HIDDEN
                for e in range(16):
                    wsplat = _take16(ws[e], kidx)
                    hb[b, i0 + e, pl.ds(h_i, HIDDEN)] = (
                        hb[b, i0 + e, pl.ds(h_i, HIDDEN)] * wsplat)
            return carry2

        lax.fori_loop(0, T // 16, edge_pair, 0)

    issue_idx(0, 0)
    issue_idx(1, 1)
    wait_idx(0)
    build_sdb(0)
    issue_gathers(0)

    wait_idx(1)
    build_sdb(1)
    issue_gathers(1)
    wait_gathers(0)
    issue_idx(2, 0)
    compute(0)
    issue_scatters(0)

    def steady(g, b):
        wait_idx(1 - b)
        wait_scatters(1 - b)
        build_sdb(1 - b)
        issue_gathers(1 - b)
        wait_gathers(b)
        issue_idx(g + 2, b)
        compute(b)
        issue_scatters(b)

    def pair(k, carry):
        steady(2 * k + 1, 1)
        steady(2 * k + 2, 0)
        return carry

    lax.fori_loop(0, (NITER - 3) // 2, pair, 0)

    wait_idx(0)
    wait_scatters(0)
    build_sdb(0)
    issue_gathers(0)
    wait_gathers(1)
    compute(1)
    issue_scatters(1)

    wait_gathers(0)
    compute(0)
    issue_scatters(0)
    wait_scatters(1)
    wait_scatters(0)

    plsc.subcore_barrier()
    pltpu.sync_copy(u_sh.at[pl.ds(r0, ROWS)], u_out.at[c, pl.ds(r0, ROWS)])
    pltpu.sync_copy(d_sh.at[pl.ds(r0, ROWS)], d_out.at[c, pl.ds(r0, ROWS)])


@functools.partial(
    pl.kernel,
    mesh=plsc.VectorSubcoreMesh(core_axis_name="c", subcore_axis_name="s"),
    compiler_params=pltpu.CompilerParams(use_tc_tiling_on_sc=False),
    out_type=[
        jax.ShapeDtypeStruct((NC, N_PAD, IN_DIM), jnp.float32),
        jax.ShapeDtypeStruct((NC, N_PAD, 2 * NUM_HEADS), jnp.float32),
    ],
    scratch_types=[
        pltpu.VMEM_SHARED((N_PAD, IN_DIM), jnp.float32),
        pltpu.VMEM_SHARED((N_PAD, 2 * NUM_HEADS), jnp.float32),
        pltpu.VMEM((2, 2 * T), jnp.int32),
        pltpu.VMEM((2, T), jnp.int32),
        pltpu.VMEM((2, T, 2 * NUM_HEADS), jnp.float32),
        pltpu.VMEM((2, T, 2 * NUM_HEADS), jnp.float32),
        pltpu.VMEM((2, T, IN_DIM), jnp.float32),
        pltpu.VMEM((2, T, 2 * NUM_HEADS), jnp.float32),
        pltpu.VMEM((16,), jnp.float32),
        pltpu.SemaphoreType.DMA,
        pltpu.SemaphoreType.DMA,
        pltpu.SemaphoreType.DMA,
        pltpu.SemaphoreType.DMA,
        pltpu.SemaphoreType.DMA,
        pltpu.SemaphoreType.DMA,
    ],
)
def _sc_edge(h_hbm, al_hbm, al2_hbm, sd_hbm, shift_hbm, z128_hbm, z8_hbm,
             u_out, d_out, *scratch):
    _sc_body(h_hbm, al_hbm, al2_hbm, sd_hbm, shift_hbm, z128_hbm, z8_hbm,
             u_out, d_out, *scratch)


def _post_body(u_ref, d_ref, k_ref, wc_ref, bc_ref, out_ref):
    usum = u_ref[0] + u_ref[1]
    dn = d_ref[0] + d_ref[1]
    den = jnp.dot(dn, k_ref[...], preferred_element_type=jnp.float32) + 1e-9
    a = usum / den
    agg = jnp.where(a > 0.0, a, jnp.exp(a) - 1.0)
    out_ref[...] = (
        jnp.dot(agg, wc_ref[...], preferred_element_type=jnp.float32)
        + bc_ref[...]
    )


def _post(U, D, K, Wc, bc):
    return pl.pallas_call(
        _post_body,
        out_shape=jax.ShapeDtypeStruct((N_PAD, OUT_DIM), jnp.float32),
    )(U, D, K, Wc, bc)


def kernel(x, edge_index, W, a_src, a_dst, Wc, bc):
    src = edge_index[0].astype(jnp.int32)
    dst = edge_index[1].astype(jnp.int32)

    eye8 = jnp.eye(NUM_HEADS, dtype=jnp.float32)
    Bs = (a_src[:, :, None] * eye8[:, None, :]).reshape(IN_DIM, NUM_HEADS)
    Bd = (a_dst[:, :, None] * eye8[:, None, :]).reshape(IN_DIM, NUM_HEADS)
    K = jnp.concatenate(
        [jnp.repeat(eye8, HIDDEN, axis=1),
         jnp.zeros((NUM_HEADS, IN_DIM), jnp.float32)], axis=0)

    h, al, al2, m = _pre(x, W, Bs, Bd)
    shift8 = jnp.maximum(m[0, :NUM_HEADS] + m[0, NUM_HEADS:], 0.0)
    shift16 = jnp.concatenate([shift8, shift8])

    z128 = jnp.zeros((N_PAD, IN_DIM), jnp.float32)
    z8 = jnp.zeros((N_PAD, 2 * NUM_HEADS), jnp.float32)

    sd = jnp.concatenate(
        [src.reshape(E // T, T), dst.reshape(E // T, T)], axis=1)

    U, D = _sc_edge(h, al, al2, sd, shift16, z128, z8)
    return _post(U, D, K, Wc, bc)[:N]


# speedup vs baseline: 2.5819x; 2.5819x over previous
import functools

import jax
import jax.numpy as jnp
from jax import lax
from jax.experimental import pallas as pl
from jax.experimental.pallas import tpu as pltpu
from jax.experimental.pallas import tpu_sc as plsc

N = 10000
E = 320000
IN_DIM = 128
NUM_HEADS = 8
HIDDEN = 16
OUT_DIM = 2

NC = 2
NS = 16
NTILES = NC * NS
EPT = E // NTILES
T = 80
NITER = EPT // T
N_PAD = 10240
ROWS = N_PAD // NS


def _pre_body(x_ref, w_ref, bs_ref, bd_ref, h_ref, al_ref, al2_ref, m_ref):
    xv = x_ref[...]
    hv = jnp.dot(xv, w_ref[...], preferred_element_type=jnp.float32)
    h_ref[...] = hv
    als = jnp.dot(hv, bs_ref[...], preferred_element_type=jnp.float32)
    ald = jnp.dot(hv, bd_ref[...], preferred_element_type=jnp.float32)
    al = jnp.concatenate([als, ald], axis=1)
    al_ref[...] = al
    al2_ref[...] = jnp.concatenate([ald, als], axis=1)
    m_ref[...] = jnp.max(al, axis=0, keepdims=True)


def _pre(x, W, Bs, Bd):
    return pl.pallas_call(
        _pre_body,
        out_shape=[
            jax.ShapeDtypeStruct((N, IN_DIM), jnp.float32),
            jax.ShapeDtypeStruct((N, 2 * NUM_HEADS), jnp.float32),
            jax.ShapeDtypeStruct((N, 2 * NUM_HEADS), jnp.float32),
            jax.ShapeDtypeStruct((1, 2 * NUM_HEADS), jnp.float32),
        ],
    )(x, W, Bs, Bd)


_GDN = lax.GatherDimensionNumbers(
    offset_dims=(), collapsed_slice_dims=(0,), start_index_map=(0,))


def _take16(v, idx):
    return lax.gather(v, idx[:, None], _GDN, (1,),
                      mode=lax.GatherScatterMode.PROMISE_IN_BOUNDS)


def _sc_body(h_hbm, al_hbm, al2_hbm, sd_hbm, shift_hbm, z128_hbm, z8_hbm,
             u_out, d_out,
             u_sh, d_sh, idxb, sdb, asb, adb, hb, wb, shb,
             si0, si1, sg0, sg1, ss0, ss1):
    c = lax.axis_index("c")
    s = lax.axis_index("s")
    tile = c * NS + s

    r0 = s * ROWS
    pltpu.sync_copy(z128_hbm.at[pl.ds(r0, ROWS)], u_sh.at[pl.ds(r0, ROWS)])
    pltpu.sync_copy(z8_hbm.at[pl.ds(r0, ROWS)], d_sh.at[pl.ds(r0, ROWS)])
    pltpu.sync_copy(shift_hbm, shb)
    plsc.subcore_barrier()

    shift = shb[...]
    sem_i = (si0, si1)
    sem_g = (sg0, sg1)
    sem_s = (ss0, ss1)
    cid0 = tile * NITER

    def issue_idx(g, b):
        pltpu.async_copy(sd_hbm.at[cid0 + g], idxb.at[b], sem_i[b])

    def wait_idx(b):
        pltpu.make_async_copy(sd_hbm.at[0], idxb.at[b], sem_i[b]).wait()

    def build_sdb(b):
        for k in range(T // 16):
            sdb[b, pl.ds(16 * k, 16)] = idxb[b, pl.ds(T + 16 * k, 16)]

    def gather_descs(b):
        srcv = idxb.at[b, pl.ds(0, T)]
        dstv = idxb.at[b, pl.ds(T, T)]
        return (
            pltpu.make_async_copy(al_hbm.at[srcv], asb.at[b], sem_g[b]),
            pltpu.make_async_copy(al2_hbm.at[dstv], adb.at[b], sem_g[b]),
            pltpu.make_async_copy(h_hbm.at[srcv], hb.at[b], sem_g[b]),
        )

    def issue_gathers(b):
        srcv = idxb.at[b, pl.ds(0, T)]
        dstv = idxb.at[b, pl.ds(T, T)]
        pltpu.async_copy(al_hbm.at[srcv], asb.at[b], sem_g[b])
        pltpu.async_copy(al2_hbm.at[dstv], adb.at[b], sem_g[b])
        pltpu.async_copy(h_hbm.at[srcv], hb.at[b], sem_g[b])

    def wait_gathers(b):
        for d in gather_descs(b):
            d.wait()

    def issue_scatters(b):
        pltpu.async_copy(hb.at[b], u_sh.at[sdb.at[b]], sem_s[b], add=True)
        pltpu.async_copy(wb.at[b], d_sh.at[sdb.at[b]], sem_s[b], add=True)

    def wait_scatters(b):
        pltpu.make_async_copy(hb.at[b], u_sh.at[sdb.at[b]], sem_s[b]).wait()
        pltpu.make_async_copy(wb.at[b], d_sh.at[sdb.at[b]], sem_s[b]).wait()

    def compute(b):
        def edge_pair(j, carry2):
            i0 = 8 * j
            ws = []
            for e in range(8):
                ra = asb[b, i0 + e, :]
                rb = adb[b, i0 + e, :]
                t = ra + rb
                t = jnp.where(t >= 0.0, t, 0.2 * t)
                w = jnp.exp(t - shift)
                wb[b, i0 + e, :] = w
                ws.append(w)
            for k in range(NUM_HEADS):
                kidx = jnp.full((16,), k, jnp.int32)
                h_i = k * HIDDEN
                for e in range(8):
                    wsplat = _take16(ws[e], kidx)
                    hb[b, i0 + e, pl.ds(h_i, HIDDEN)] = (
                        hb[b, i0 + e, pl.ds(h_i, HIDDEN)] * wsplat)
            return carry2

        lax.fori_loop(0, T // 8, edge_pair, 0)

    issue_idx(0, 0)
    issue_idx(1, 1)
    wait_idx(0)
    build_sdb(0)
    issue_gathers(0)

    wait_idx(1)
    build_sdb(1)
    issue_gathers(1)
    wait_gathers(0)
    issue_idx(2, 0)
    compute(0)
    issue_scatters(0)

    def steady(g, b):
        wait_idx(1 - b)
        wait_scatters(1 - b)
        build_sdb(1 - b)
        issue_gathers(1 - b)
        wait_gathers(b)
        issue_idx(g + 2, b)
        compute(b)
        issue_scatters(b)

    def pair(k, carry):
        steady(2 * k + 1, 1)
        steady(2 * k + 2, 0)
        return carry

    lax.fori_loop(0, (NITER - 3) // 2, pair, 0)

    wait_idx(0)
    wait_scatters(0)
    build_sdb(0)
    issue_gathers(0)
    wait_gathers(1)
    compute(1)
    issue_scatters(1)

    wait_gathers(0)
    compute(0)
    issue_scatters(0)
    wait_scatters(1)
    wait_scatters(0)

    plsc.subcore_barrier()
    pltpu.sync_copy(u_sh.at[pl.ds(r0, ROWS)], u_out.at[c, pl.ds(r0, ROWS)])
    pltpu.sync_copy(d_sh.at[pl.ds(r0, ROWS)], d_out.at[c, pl.ds(r0, ROWS)])


@functools.partial(
    pl.kernel,
    mesh=plsc.VectorSubcoreMesh(core_axis_name="c", subcore_axis_name="s"),
    compiler_params=pltpu.CompilerParams(use_tc_tiling_on_sc=False),
    out_type=[
        jax.ShapeDtypeStruct((NC, N_PAD, IN_DIM), jnp.float32),
        jax.ShapeDtypeStruct((NC, N_PAD, 2 * NUM_HEADS), jnp.float32),
    ],
    scratch_types=[
        pltpu.VMEM_SHARED((N_PAD, IN_DIM), jnp.float32),
        pltpu.VMEM_SHARED((N_PAD, 2 * NUM_HEADS), jnp.float32),
        pltpu.VMEM((2, 2 * T), jnp.int32),
        pltpu.VMEM((2, T), jnp.int32),
        pltpu.VMEM((2, T, 2 * NUM_HEADS), jnp.float32),
        pltpu.VMEM((2, T, 2 * NUM_HEADS), jnp.float32),
        pltpu.VMEM((2, T, IN_DIM), jnp.float32),
        pltpu.VMEM((2, T, 2 * NUM_HEADS), jnp.float32),
        pltpu.VMEM((16,), jnp.float32),
        pltpu.SemaphoreType.DMA,
        pltpu.SemaphoreType.DMA,
        pltpu.SemaphoreType.DMA,
        pltpu.SemaphoreType.DMA,
        pltpu.SemaphoreType.DMA,
        pltpu.SemaphoreType.DMA,
    ],
)
def _sc_edge(h_hbm, al_hbm, al2_hbm, sd_hbm, shift_hbm, z128_hbm, z8_hbm,
             u_out, d_out, *scratch):
    _sc_body(h_hbm, al_hbm, al2_hbm, sd_hbm, shift_hbm, z128_hbm, z8_hbm,
             u_out, d_out, *scratch)


def _post_body(u_ref, d_ref, k_ref, wc_ref, bc_ref, out_ref):
    usum = u_ref[0] + u_ref[1]
    dn = d_ref[0] + d_ref[1]
    den = jnp.dot(dn, k_ref[...], preferred_element_type=jnp.float32) + 1e-9
    a = usum / den
    agg = jnp.where(a > 0.0, a, jnp.exp(a) - 1.0)
    out_ref[...] = (
        jnp.dot(agg, wc_ref[...], preferred_element_type=jnp.float32)
        + bc_ref[...]
    )


def _post(U, D, K, Wc, bc):
    return pl.pallas_call(
        _post_body,
        out_shape=jax.ShapeDtypeStruct((N_PAD, OUT_DIM), jnp.float32),
    )(U, D, K, Wc, bc)


def kernel(x, edge_index, W, a_src, a_dst, Wc, bc):
    src = edge_index[0].astype(jnp.int32)
    dst = edge_index[1].astype(jnp.int32)

    eye8 = jnp.eye(NUM_HEADS, dtype=jnp.float32)
    Bs = (a_src[:, :, None] * eye8[:, None, :]).reshape(IN_DIM, NUM_HEADS)
    Bd = (a_dst[:, :, None] * eye8[:, None, :]).reshape(IN_DIM, NUM_HEADS)
    K = jnp.concatenate(
        [jnp.repeat(eye8, HIDDEN, axis=1),
         jnp.zeros((NUM_HEADS, IN_DIM), jnp.float32)], axis=0)

    h, al, al2, m = _pre(x, W, Bs, Bd)
    shift8 = jnp.maximum(m[0, :NUM_HEADS] + m[0, NUM_HEADS:], 0.0)
    shift16 = jnp.concatenate([shift8, shift8])

    z128 = jnp.zeros((N_PAD, IN_DIM), jnp.float32)
    z8 = jnp.zeros((N_PAD, 2 * NUM_HEADS), jnp.float32)

    sd = jnp.concatenate(
        [src.reshape(E // T, T), dst.reshape(E // T, T)], axis=1)

    U, D = _sc_edge(h, al, al2, sd, shift16, z128, z8)
    return _post(U, D, K, Wc, bc)[:N]
